# Initial kernel scaffold; baseline (speedup 1.0000x reference)
import jax
import jax.numpy as jnp
from jax.experimental import pallas as pl

_B, _D, _N = 8, 256, 2048
_Q, _K = 8, 1024
_TN = 512


def _rvq_body(x_ref, cb_ref, cn_ref, out_ref):
    r = x_ref[0]
    acc = jnp.zeros_like(r)
    iota_k = jax.lax.broadcasted_iota(jnp.int32, (_K, _TN), 0)
    for q in range(_Q):
        cb = cb_ref[q]
        scores = jax.lax.dot_general(
            cb, r, (((1,), (0,)), ((), ())),
            precision=jax.lax.Precision.HIGHEST,
            preferred_element_type=jnp.float32,
        ) - cn_ref[q]
        m = jnp.max(scores, axis=0, keepdims=True)
        masked = jnp.where(scores == m, iota_k, _K)
        best = jnp.min(masked, axis=0, keepdims=True)
        onehot = (iota_k == best).astype(jnp.float32)
        quant = jax.lax.dot_general(
            cb, onehot, (((0,), (0,)), ((), ())),
            precision=jax.lax.Precision.HIGHEST,
            preferred_element_type=jnp.float32,
        )
        acc = acc + quant
        r = r - quant
    out_ref[0] = acc


@jax.jit
def kernel(x, codebooks):
    cn = 0.5 * jnp.sum(codebooks * codebooks, axis=-1, keepdims=True)
    grid = (_B, _N // _TN)
    return pl.pallas_call(
        _rvq_body,
        grid=grid,
        in_specs=[
            pl.BlockSpec((1, _D, _TN), lambda b, n: (b, 0, n)),
            pl.BlockSpec((_Q, _K, _D), lambda b, n: (0, 0, 0)),
            pl.BlockSpec((_Q, _K, 1), lambda b, n: (0, 0, 0)),
        ],
        out_specs=pl.BlockSpec((1, _D, _TN), lambda b, n: (b, 0, n)),
        out_shape=jax.ShapeDtypeStruct((_B, _D, _N), jnp.float32),
    )(x, codebooks, cn)


# speedup vs baseline: 1.3803x; 1.3803x over previous
import jax
import jax.numpy as jnp
from jax.experimental import pallas as pl

_B, _D, _N = 8, 256, 2048
_Q, _K = 8, 1024
_TN = 512


def _rvq_body(x_ref, cb_ref, cn_ref, out_ref):
    r = x_ref[0]
    acc = jnp.zeros_like(r)
    iota_k = jax.lax.broadcasted_iota(jnp.int32, (_K, _TN), 0)
    for q in range(_Q):
        cb = cb_ref[q]
        scores = jax.lax.dot_general(
            cb, r, (((1,), (0,)), ((), ())),
            precision=jax.lax.Precision.DEFAULT,
            preferred_element_type=jnp.float32,
        ) - cn_ref[q]
        m = jnp.max(scores, axis=0, keepdims=True)
        masked = jnp.where(scores == m, iota_k, _K)
        best = jnp.min(masked, axis=0, keepdims=True)
        onehot = (iota_k == best).astype(jnp.float32)
        quant = jax.lax.dot_general(
            cb, onehot, (((0,), (0,)), ((), ())),
            precision=jax.lax.Precision.HIGHEST,
            preferred_element_type=jnp.float32,
        )
        acc = acc + quant
        r = r - quant
    out_ref[0] = acc


@jax.jit
def kernel(x, codebooks):
    cn = 0.5 * jnp.sum(codebooks * codebooks, axis=-1, keepdims=True)
    grid = (_B, _N // _TN)
    return pl.pallas_call(
        _rvq_body,
        grid=grid,
        in_specs=[
            pl.BlockSpec((1, _D, _TN), lambda b, n: (b, 0, n)),
            pl.BlockSpec((_Q, _K, _D), lambda b, n: (0, 0, 0)),
            pl.BlockSpec((_Q, _K, 1), lambda b, n: (0, 0, 0)),
        ],
        out_specs=pl.BlockSpec((1, _D, _TN), lambda b, n: (b, 0, n)),
        out_shape=jax.ShapeDtypeStruct((_B, _D, _N), jnp.float32),
    )(x, codebooks, cn)


# speedup vs baseline: 2.3317x; 1.6892x over previous
import jax
import jax.numpy as jnp
from jax.experimental import pallas as pl

_B, _D, _N = 8, 256, 2048
_Q, _K = 8, 1024
_TN = 512


_H = _TN // 2


def _rvq_body(x_ref, cb_ref, cbp_ref, cn_ref, out_ref):
    x = x_ref[0]
    rs = [x[:, :_H], x[:, _H:]]
    ni = -jax.lax.broadcasted_iota(jnp.int32, (_K, _H), 0).astype(jnp.float32)
    for q in range(_Q):
        cb = cb_ref[q]
        cbp = cbp_ref[q]
        cn = cn_ref[q]
        for h in range(2):
            r = rs[h]
            scores = jax.lax.dot_general(
                cb, r, (((1,), (0,)), ((), ())),
                precision=jax.lax.Precision.DEFAULT,
                preferred_element_type=jnp.float32,
            ) - cn
            m = jnp.max(scores, axis=0, keepdims=True)
            masked = jnp.where(scores == m, ni, -jnp.inf)
            bestn = jnp.max(masked, axis=0, keepdims=True)
            onehot = (ni == bestn).astype(jnp.bfloat16)
            q3 = jax.lax.dot_general(
                cbp, onehot, (((0,), (0,)), ((), ())),
                precision=jax.lax.Precision.DEFAULT,
                preferred_element_type=jnp.float32,
            )
            quant = (q3[0:_D] + q3[_D:2 * _D]) + q3[2 * _D:3 * _D]
            rs[h] = r - quant
    out_ref[0] = x - jnp.concatenate(rs, axis=1)


@jax.jit
def kernel(x, codebooks):
    cn = 0.5 * jnp.sum(codebooks * codebooks, axis=-1, keepdims=True)
    hi = codebooks.astype(jnp.bfloat16)
    r1 = codebooks - hi.astype(jnp.float32)
    mid = r1.astype(jnp.bfloat16)
    lo = (r1 - mid.astype(jnp.float32)).astype(jnp.bfloat16)
    cbp = jnp.concatenate([hi, mid, lo], axis=-1)
    grid = (_B, _N // _TN)
    return pl.pallas_call(
        _rvq_body,
        grid=grid,
        in_specs=[
            pl.BlockSpec((1, _D, _TN), lambda b, n: (b, 0, n)),
            pl.BlockSpec((_Q, _K, _D), lambda b, n: (0, 0, 0)),
            pl.BlockSpec((_Q, _K, 3 * _D), lambda b, n: (0, 0, 0)),
            pl.BlockSpec((_Q, _K, 1), lambda b, n: (0, 0, 0)),
        ],
        out_specs=pl.BlockSpec((1, _D, _TN), lambda b, n: (b, 0, n)),
        out_shape=jax.ShapeDtypeStruct((_B, _D, _N), jnp.float32),
    )(x, codebooks, cbp, cn)
